# baseline (speedup 1.0000x reference)
import functools

import jax
import jax.numpy as jnp
from jax import lax
from jax.experimental import pallas as pl
from jax.experimental.pallas import tpu as pltpu
from jax.experimental.pallas import tpu_sc as plsc

B = 1024
CIN = 128
L = 256
D = 64
K = 8192
BETA = 0.25

BT = 32
BTV = 128


def _lrelu(h):
    return jnp.maximum(h, 0.001 * h)


def _encoder_kernel(x_ref, w1_ref, b1_ref, w2_ref, b2_ref, w3_ref, b3_ref, z_ref):
    xt = jnp.transpose(x_ref[...], (0, 2, 1))
    xm = xt.reshape(BT * L, CIN)
    h1 = jax.lax.dot_general(
        xm, w1_ref[...], (((1,), (1,)), ((), ())),
        preferred_element_type=jnp.float32)
    h1 = _lrelu(h1 + b1_ref[...])
    h2 = jax.lax.dot_general(
        h1, w2_ref[...], (((1,), (1,)), ((), ())),
        preferred_element_type=jnp.float32)
    h2 = _lrelu(h2 + b2_ref[...])
    h3 = jax.lax.dot_general(
        h2, w3_ref[...], (((1,), (1,)), ((), ())),
        preferred_element_type=jnp.float32)
    h3 = h3 + b3_ref[...]
    z_ref[...] = jnp.max(h3.reshape(BT, L, D), axis=1)


def _encode(x, ew1, eb1, ew2, eb2, ew3, eb3):
    return pl.pallas_call(
        _encoder_kernel,
        grid=(B // BT,),
        in_specs=[
            pl.BlockSpec((BT, CIN, L), lambda i: (i, 0, 0)),
            pl.BlockSpec((64, CIN), lambda i: (0, 0)),
            pl.BlockSpec((1, 64), lambda i: (0, 0)),
            pl.BlockSpec((128, 64), lambda i: (0, 0)),
            pl.BlockSpec((1, 128), lambda i: (0, 0)),
            pl.BlockSpec((D, 128), lambda i: (0, 0)),
            pl.BlockSpec((1, D), lambda i: (0, 0)),
        ],
        out_specs=pl.BlockSpec((BT, D), lambda i: (i, 0)),
        out_shape=jax.ShapeDtypeStruct((B, D), jnp.float32),
    )(x, ew1, eb1.reshape(1, 64), ew2, eb2.reshape(1, 128), ew3, eb3.reshape(1, D))


def _vq_kernel(z_ref, zn_ref, cn_ref, cb_ref, idx_ref):
    c = jax.lax.dot_general(
        z_ref[...], cb_ref[...], (((1,), (1,)), ((), ())),
        preferred_element_type=jnp.float32)
    dmat = (zn_ref[...] + cn_ref[...]) - 2.0 * c
    m = jnp.min(dmat, axis=1, keepdims=True)
    io = lax.broadcasted_iota(jnp.int32, (BTV, K), 1)
    first = jnp.min(jnp.where(dmat == m, io, K), axis=1)
    idx_ref[0, 0, :] = first


def _vq_argmin(z, zn, cn, codebook):
    out = pl.pallas_call(
        _vq_kernel,
        grid=(B // BTV,),
        in_specs=[
            pl.BlockSpec((BTV, D), lambda i: (i, 0)),
            pl.BlockSpec((BTV, 1), lambda i: (i, 0)),
            pl.BlockSpec((1, K), lambda i: (0, 0)),
            pl.BlockSpec((K, D), lambda i: (0, 0)),
        ],
        out_specs=pl.BlockSpec((1, 1, BTV), lambda i: (i, 0, 0)),
        out_shape=jax.ShapeDtypeStruct((B // BTV, 1, BTV), jnp.int32),
    )(z, zn, cn, codebook)
    return out.reshape(B)


def _sc_gather(codebook, idx):
    try:
        info = plsc.get_sparse_core_info()
        nc, ns = info.num_cores, info.num_subcores
    except Exception:
        nc, ns = 2, 16
    nw = nc * ns
    b_per_w = B // nw
    dpad = 128
    table = jnp.concatenate(
        [codebook, jnp.zeros((K, dpad - D), jnp.float32)], axis=1)
    mesh = plsc.VectorSubcoreMesh(core_axis_name="c", subcore_axis_name="s")

    @functools.partial(
        pl.kernel, mesh=mesh,
        out_type=jax.ShapeDtypeStruct((B, dpad), jnp.float32),
        scratch_types=[
            pltpu.VMEM((b_per_w,), jnp.int32),
            pltpu.VMEM((b_per_w, dpad), jnp.float32),
            pltpu.SemaphoreType.DMA,
        ],
    )
    def gk(table_hbm, idx_hbm, out_hbm, idx_v, rows_v, sem):
        wid = lax.axis_index("s") * nc + lax.axis_index("c")
        base = wid * b_per_w
        pltpu.sync_copy(idx_hbm.at[pl.ds(base, b_per_w)], idx_v)
        pltpu.async_copy(table_hbm.at[idx_v], rows_v, sem).wait()
        pltpu.sync_copy(rows_v, out_hbm.at[pl.ds(base, b_per_w)])

    return gk(table, idx)


def _decoder_kernel(z_ref, zq_ref, w1_ref, b1_ref, w2_ref, b2_ref, w3_ref,
                    b3_ref, xr_ref, loss_ref):
    z = z_ref[...]
    zq = zq_ref[:, :D]
    diff = zq - z
    m = jnp.sum(diff * diff) * (1.0 / (B * D))
    loss_ref[...] = jnp.reshape(m + BETA * m, (1, 1))
    zst = z + diff
    h1 = jax.lax.dot_general(
        zst, w1_ref[...], (((1,), (0,)), ((), ())),
        preferred_element_type=jnp.float32)
    h1 = _lrelu(h1 + b1_ref[...])
    h2 = jax.lax.dot_general(
        h1, w2_ref[...], (((1,), (0,)), ((), ())),
        preferred_element_type=jnp.float32)
    h2 = _lrelu(h2 + b2_ref[...])
    xr = jax.lax.dot_general(
        h2, w3_ref[...], (((1,), (0,)), ((), ())),
        preferred_element_type=jnp.float32)
    xr_ref[...] = xr + b3_ref[...]


def _decode(z, zq, dw1, db1, dw2, db2, dw3, db3):
    xr, loss = pl.pallas_call(
        _decoder_kernel,
        grid=(1,),
        in_specs=[
            pl.BlockSpec((B, D), lambda i: (0, 0)),
            pl.BlockSpec((B, 128), lambda i: (0, 0)),
            pl.BlockSpec((D, 128), lambda i: (0, 0)),
            pl.BlockSpec((1, 128), lambda i: (0, 0)),
            pl.BlockSpec((128, 64), lambda i: (0, 0)),
            pl.BlockSpec((1, 64), lambda i: (0, 0)),
            pl.BlockSpec((64, CIN), lambda i: (0, 0)),
            pl.BlockSpec((1, CIN), lambda i: (0, 0)),
        ],
        out_specs=[
            pl.BlockSpec((B, CIN), lambda i: (0, 0)),
            pl.BlockSpec((1, 1), lambda i: (0, 0)),
        ],
        out_shape=[
            jax.ShapeDtypeStruct((B, CIN), jnp.float32),
            jax.ShapeDtypeStruct((1, 1), jnp.float32),
        ],
    )(z, zq, dw1, db1.reshape(1, 128), dw2, db2.reshape(1, 64),
      dw3, db3.reshape(1, CIN))
    return xr, loss


def kernel(x, ew1, eb1, ew2, eb2, ew3, eb3, codebook, dw1, db1, dw2, db2, dw3, db3):
    z = _encode(x, ew1, eb1, ew2, eb2, ew3, eb3)
    zn = jnp.sum(z ** 2, axis=1, keepdims=True)
    cn = jnp.sum(codebook ** 2, axis=1).reshape(1, K)
    idx = _vq_argmin(z, zn, cn, codebook)
    zq = _sc_gather(codebook, idx)
    xr, loss = _decode(z, zq, dw1, db1, dw2, db2, dw3, db3)
    return (xr.reshape(B, CIN, 1), loss.reshape(()), z, D)


# speedup vs baseline: 1.0229x; 1.0229x over previous
import functools

import jax
import jax.numpy as jnp
from jax import lax
from jax.experimental import pallas as pl
from jax.experimental.pallas import tpu as pltpu
from jax.experimental.pallas import tpu_sc as plsc

B = 1024
CIN = 128
L = 256
D = 64
K = 8192
BETA = 0.25

BT = 32
BTV = 128


def _lrelu(h):
    return jnp.where(h >= 0, h, 0.001 * h)


def _encoder_kernel(x_ref, w1_ref, b1_ref, w2_ref, b2_ref, w3_ref, b3_ref, z_ref):
    xt = jnp.transpose(x_ref[...], (0, 2, 1))
    xm = xt.reshape(BT * L, CIN)
    h1 = jax.lax.dot_general(
        xm, w1_ref[...], (((1,), (1,)), ((), ())),
        preferred_element_type=jnp.float32)
    h1 = _lrelu(h1 + b1_ref[...])
    h2 = jax.lax.dot_general(
        h1, w2_ref[...], (((1,), (1,)), ((), ())),
        preferred_element_type=jnp.float32)
    h2 = _lrelu(h2 + b2_ref[...])
    h3 = jax.lax.dot_general(
        h2, w3_ref[...], (((1,), (1,)), ((), ())),
        preferred_element_type=jnp.float32)
    h3 = h3 + b3_ref[...]
    z_ref[...] = jnp.max(h3.reshape(BT, L, D), axis=1)


def _encode(x, ew1, eb1, ew2, eb2, ew3, eb3):
    return pl.pallas_call(
        _encoder_kernel,
        grid=(B // BT,),
        in_specs=[
            pl.BlockSpec((BT, CIN, L), lambda i: (i, 0, 0)),
            pl.BlockSpec((64, CIN), lambda i: (0, 0)),
            pl.BlockSpec((1, 64), lambda i: (0, 0)),
            pl.BlockSpec((128, 64), lambda i: (0, 0)),
            pl.BlockSpec((1, 128), lambda i: (0, 0)),
            pl.BlockSpec((D, 128), lambda i: (0, 0)),
            pl.BlockSpec((1, D), lambda i: (0, 0)),
        ],
        out_specs=pl.BlockSpec((BT, D), lambda i: (i, 0)),
        out_shape=jax.ShapeDtypeStruct((B, D), jnp.float32),
    )(x, ew1, eb1.reshape(1, 64), ew2, eb2.reshape(1, 128), ew3, eb3.reshape(1, D))


def _vq_kernel(z_ref, zn_ref, cn_ref, cb_ref, idx_ref):
    c = jax.lax.dot_general(
        z_ref[...], cb_ref[...], (((1,), (1,)), ((), ())),
        preferred_element_type=jnp.float32)
    dmat = (zn_ref[...] + cn_ref[...]) - 2.0 * c
    m = jnp.min(dmat, axis=1, keepdims=True)
    io = lax.broadcasted_iota(jnp.int32, (BTV, K), 1)
    first = jnp.min(jnp.where(dmat == m, io, K), axis=1)
    idx_ref[0, 0, :] = first


def _vq_argmin(z, zn, cn, codebook):
    out = pl.pallas_call(
        _vq_kernel,
        grid=(B // BTV,),
        in_specs=[
            pl.BlockSpec((BTV, D), lambda i: (i, 0)),
            pl.BlockSpec((BTV, 1), lambda i: (i, 0)),
            pl.BlockSpec((1, K), lambda i: (0, 0)),
            pl.BlockSpec((K, D), lambda i: (0, 0)),
        ],
        out_specs=pl.BlockSpec((1, 1, BTV), lambda i: (i, 0, 0)),
        out_shape=jax.ShapeDtypeStruct((B // BTV, 1, BTV), jnp.int32),
    )(z, zn, cn, codebook)
    return out.reshape(B)


def _sc_gather(codebook, idx):
    try:
        info = plsc.get_sparse_core_info()
        nc, ns = info.num_cores, info.num_subcores
    except Exception:
        nc, ns = 2, 16
    nw = nc * ns
    b_per_w = B // nw
    dpad = 128
    table = jnp.concatenate(
        [codebook, jnp.zeros((K, dpad - D), jnp.float32)], axis=1)
    mesh = plsc.VectorSubcoreMesh(core_axis_name="c", subcore_axis_name="s")
    nchunk = 4
    rpc = b_per_w // nchunk

    @functools.partial(
        pl.kernel, mesh=mesh,
        out_type=jax.ShapeDtypeStruct((B, dpad), jnp.float32),
        scratch_types=[
            pltpu.VMEM((b_per_w,), jnp.int32),
            pltpu.VMEM((b_per_w, dpad), jnp.float32),
            pltpu.SemaphoreType.DMA,
        ],
    )
    def gk(table_hbm, idx_hbm, out_hbm, idx_v, rows_v, sem):
        wid = lax.axis_index("s") * nc + lax.axis_index("c")
        base = wid * b_per_w
        pltpu.sync_copy(idx_hbm.at[pl.ds(base, b_per_w)], idx_v)
        handles = [
            pltpu.async_copy(
                table_hbm.at[idx_v.at[pl.ds(c * rpc, rpc)]],
                rows_v.at[pl.ds(c * rpc, rpc)],
                sem)
            for c in range(nchunk)]
        for h in handles:
            h.wait()
        pltpu.sync_copy(rows_v, out_hbm.at[pl.ds(base, b_per_w)])

    return gk(table, idx)


def _decoder_kernel(z_ref, zq_ref, w1_ref, b1_ref, w2_ref, b2_ref, w3_ref,
                    b3_ref, xr_ref, loss_ref):
    z = z_ref[...]
    zq = zq_ref[:, :D]
    diff = zq - z
    m = jnp.sum(diff * diff) * (1.0 / (B * D))
    loss_ref[...] = jnp.reshape(m + BETA * m, (1, 1))
    zst = z + diff
    h1 = jax.lax.dot_general(
        zst, w1_ref[...], (((1,), (0,)), ((), ())),
        preferred_element_type=jnp.float32)
    h1 = _lrelu(h1 + b1_ref[...])
    h2 = jax.lax.dot_general(
        h1, w2_ref[...], (((1,), (0,)), ((), ())),
        preferred_element_type=jnp.float32)
    h2 = _lrelu(h2 + b2_ref[...])
    xr = jax.lax.dot_general(
        h2, w3_ref[...], (((1,), (0,)), ((), ())),
        preferred_element_type=jnp.float32)
    xr_ref[...] = xr + b3_ref[...]


def _decode(z, zq, dw1, db1, dw2, db2, dw3, db3):
    xr, loss = pl.pallas_call(
        _decoder_kernel,
        grid=(1,),
        in_specs=[
            pl.BlockSpec((B, D), lambda i: (0, 0)),
            pl.BlockSpec((B, 128), lambda i: (0, 0)),
            pl.BlockSpec((D, 128), lambda i: (0, 0)),
            pl.BlockSpec((1, 128), lambda i: (0, 0)),
            pl.BlockSpec((128, 64), lambda i: (0, 0)),
            pl.BlockSpec((1, 64), lambda i: (0, 0)),
            pl.BlockSpec((64, CIN), lambda i: (0, 0)),
            pl.BlockSpec((1, CIN), lambda i: (0, 0)),
        ],
        out_specs=[
            pl.BlockSpec((B, CIN), lambda i: (0, 0)),
            pl.BlockSpec((1, 1), lambda i: (0, 0)),
        ],
        out_shape=[
            jax.ShapeDtypeStruct((B, CIN), jnp.float32),
            jax.ShapeDtypeStruct((1, 1), jnp.float32),
        ],
    )(z, zq, dw1, db1.reshape(1, 128), dw2, db2.reshape(1, 64),
      dw3, db3.reshape(1, CIN))
    return xr, loss


def kernel(x, ew1, eb1, ew2, eb2, ew3, eb3, codebook, dw1, db1, dw2, db2, dw3, db3):
    z = _encode(x, ew1, eb1, ew2, eb2, ew3, eb3)
    zn = jnp.sum(z ** 2, axis=1, keepdims=True)
    cn = jnp.sum(codebook ** 2, axis=1).reshape(1, K)
    idx = _vq_argmin(z, zn, cn, codebook)
    zq = _sc_gather(codebook, idx)
    xr, loss = _decode(z, zq, dw1, db1, dw2, db2, dw3, db3)
    return (xr.reshape(B, CIN, 1), loss.reshape(()), z, D)
